# Initial kernel scaffold; baseline (speedup 1.0000x reference)
import jax
import jax.numpy as jnp
from jax.experimental import pallas as pl


def kernel(x, kohonen_weights, grossberg_weights):
    raise NotImplementedError("write your pallas kernel here")


# speedup vs baseline: 7.5199x; 7.5199x over previous
import functools

import jax
import jax.numpy as jnp
from jax import lax
from jax.experimental import pallas as pl
from jax.experimental.pallas import tpu as pltpu
from jax.experimental.pallas import tpu_sc as plsc

_B = 16384
_D = 32
_H = 8192
_O = 64

_BB = 512


_HC = _H // 2


def _round_bf16(v):
    u = lax.bitcast_convert_type(v, jnp.uint32)
    u = (u + 0x7FFF + ((u >> 16) & 1)) & jnp.uint32(0xFFFF0000)
    return lax.bitcast_convert_type(u, jnp.float32)


def _winners_body(x_ref, w_ref, xsq_ref, wsq_ref, win_ref):
    xb16 = x_ref[...].astype(jnp.bfloat16)
    x_sq = xsq_ref[...]
    mins, idxs = [], []
    for h in range(2):
        wc = w_ref[pl.ds(h * _HC, _HC), :]
        w_sq = wsq_ref[:, pl.ds(h * _HC, _HC)]
        mm = lax.dot_general(xb16, wc.astype(jnp.bfloat16),
                             (((1,), (1,)), ((), ())),
                             preferred_element_type=jnp.float32)
        d2 = (x_sq + w_sq) - 2.0 * mm
        dist = jnp.sqrt(jnp.maximum(d2, 0.0))
        m = jnp.min(dist, axis=1, keepdims=True)
        iota = lax.broadcasted_iota(jnp.int32, (_BB, _HC), 1) + h * _HC
        idx = jnp.min(jnp.where(dist == m, iota, _H), axis=1, keepdims=True)
        mins.append(m)
        idxs.append(idx)
    take1 = mins[1] < _round_bf16(mins[0])
    win_ref[...] = jnp.where(take1, idxs[1], idxs[0])


def _compute_winners(x, kohonen_weights, x_sq, w_sq):
    grid = (_B // _BB,)
    return pl.pallas_call(
        _winners_body,
        grid=grid,
        in_specs=[
            pl.BlockSpec((_BB, _D), lambda i: (i, 0)),
            pl.BlockSpec((_H, _D), lambda i: (0, 0)),
            pl.BlockSpec((_BB, 1), lambda i: (i, 0)),
            pl.BlockSpec((1, _H), lambda i: (0, 0)),
        ],
        out_specs=pl.BlockSpec((_BB, 1), lambda i: (i, 0)),
        out_shape=jax.ShapeDtypeStruct((_B, 1), jnp.int32),
    )(x, kohonen_weights, x_sq, w_sq)


def _make_gather():
    info = plsc.get_sparse_core_info()
    nc, ns = info.num_cores, info.num_subcores
    nw = nc * ns
    b_per_w = _B // nw
    n_chunks = b_per_w // 128
    mesh = plsc.VectorSubcoreMesh(core_axis_name="c", subcore_axis_name="s")

    @functools.partial(
        pl.kernel,
        mesh=mesh,
        out_type=jax.ShapeDtypeStruct((_B, 128), jnp.float32),
        scratch_types=[
            pltpu.VMEM((n_chunks, 128), jnp.int32),
            pltpu.VMEM((b_per_w, 128), jnp.float32),
            pltpu.SemaphoreType.DMA,
        ],
    )
    def gather_k(table_hbm, idx_hbm, out_hbm, idx_v, rows_v, sem):
        wid = lax.axis_index("s") * nc + lax.axis_index("c")
        base = wid * b_per_w
        pltpu.sync_copy(idx_hbm.at[pl.ds(wid * n_chunks, n_chunks)], idx_v)
        copies = [
            pltpu.async_copy(table_hbm.at[idx_v.at[j]],
                             rows_v.at[pl.ds(j * 128, 128)], sem)
            for j in range(n_chunks)
        ]
        for c in copies:
            c.wait()
        pltpu.sync_copy(rows_v, out_hbm.at[pl.ds(base, b_per_w)])

    return gather_k, n_chunks


def kernel(x, kohonen_weights, grossberg_weights):
    b = x.shape[0]
    xf = x.reshape(b, -1)
    x_sq = jnp.sum(xf * xf, axis=1, keepdims=True)
    w_sq = jnp.sum(kohonen_weights * kohonen_weights, axis=1)

    winners2d = _compute_winners(xf, kohonen_weights, x_sq,
                                 w_sq.reshape(1, _H))
    winners = winners2d.reshape(_B)

    gather_k, n_chunks = _make_gather()
    table = jnp.pad(grossberg_weights.T, ((0, 0), (0, 128 - _O)))
    idx2d = winners.reshape(_B // 128, 128)
    output = gather_k(table, idx2d)[:, :_O]
    output = output.astype(jnp.bfloat16).astype(jnp.float32)
    return (output, winners, b)


# speedup vs baseline: 10.0970x; 1.3427x over previous
import functools

import jax
import jax.numpy as jnp
from jax import lax
from jax.experimental import pallas as pl
from jax.experimental.pallas import tpu as pltpu
from jax.experimental.pallas import tpu_sc as plsc

_B = 16384
_D = 32
_H = 8192
_O = 64

_BB = 512


_HC = _H // 2


def _round_bf16(v):
    u = lax.bitcast_convert_type(v, jnp.uint32)
    u = (u + 0x7FFF + ((u >> 16) & 1)) & jnp.uint32(0xFFFF0000)
    return lax.bitcast_convert_type(u, jnp.float32)


def _winners_body(x_ref, w_ref, xsq_ref, wsq_ref, win_ref):
    xb16 = x_ref[...].astype(jnp.bfloat16)
    x_sq = xsq_ref[...]
    mins, idxs = [], []
    for h in range(2):
        wc = w_ref[pl.ds(h * _HC, _HC), :]
        w_sq = wsq_ref[:, pl.ds(h * _HC, _HC)]
        mm = lax.dot_general(xb16, wc.astype(jnp.bfloat16),
                             (((1,), (1,)), ((), ())),
                             preferred_element_type=jnp.float32)
        d2 = (x_sq + w_sq) - 2.0 * mm
        m2 = jnp.maximum(jnp.min(d2, axis=1, keepdims=True), 0.0)
        s = jnp.sqrt(m2)
        m2_bits = lax.bitcast_convert_type(m2, jnp.int32)
        hi = m2
        for k in range(1, 8):
            cand = lax.bitcast_convert_type(m2_bits + k, jnp.float32)
            hi = jnp.where(jnp.sqrt(cand) == s, cand, hi)
        iota = lax.broadcasted_iota(jnp.int32, (_BB, _HC), 1) + h * _HC
        idx = jnp.min(jnp.where(d2 <= hi, iota, _H), axis=1, keepdims=True)
        mins.append(s)
        idxs.append(idx)
    take1 = mins[1] < _round_bf16(mins[0])
    win_ref[...] = jnp.where(take1, idxs[1], idxs[0])


def _compute_winners(x, kohonen_weights, x_sq, w_sq):
    grid = (_B // _BB,)
    return pl.pallas_call(
        _winners_body,
        grid=grid,
        in_specs=[
            pl.BlockSpec((_BB, _D), lambda i: (i, 0)),
            pl.BlockSpec((_H, _D), lambda i: (0, 0)),
            pl.BlockSpec((_BB, 1), lambda i: (i, 0)),
            pl.BlockSpec((1, _H), lambda i: (0, 0)),
        ],
        out_specs=pl.BlockSpec((_BB, 1), lambda i: (i, 0)),
        out_shape=jax.ShapeDtypeStruct((_B, 1), jnp.int32),
    )(x, kohonen_weights, x_sq, w_sq)


def _make_gather():
    info = plsc.get_sparse_core_info()
    nc, ns = info.num_cores, info.num_subcores
    nw = nc * ns
    b_per_w = _B // nw
    n_chunks = b_per_w // 128
    mesh = plsc.VectorSubcoreMesh(core_axis_name="c", subcore_axis_name="s")

    @functools.partial(
        pl.kernel,
        mesh=mesh,
        out_type=jax.ShapeDtypeStruct((_B, 128), jnp.float32),
        scratch_types=[
            pltpu.VMEM((n_chunks, 128), jnp.int32),
            pltpu.VMEM((b_per_w, 128), jnp.float32),
            pltpu.SemaphoreType.DMA,
        ],
    )
    def gather_k(table_hbm, idx_hbm, out_hbm, idx_v, rows_v, sem):
        wid = lax.axis_index("s") * nc + lax.axis_index("c")
        base = wid * b_per_w
        pltpu.sync_copy(idx_hbm.at[pl.ds(wid * n_chunks, n_chunks)], idx_v)
        copies = [
            pltpu.async_copy(table_hbm.at[idx_v.at[j]],
                             rows_v.at[pl.ds(j * 128, 128)], sem)
            for j in range(n_chunks)
        ]
        for c in copies:
            c.wait()
        pltpu.sync_copy(rows_v, out_hbm.at[pl.ds(base, b_per_w)])

    return gather_k, n_chunks


def kernel(x, kohonen_weights, grossberg_weights):
    b = x.shape[0]
    xf = x.reshape(b, -1)
    x_sq = jnp.sum(xf * xf, axis=1, keepdims=True)
    w_sq = jnp.sum(kohonen_weights * kohonen_weights, axis=1)

    winners2d = _compute_winners(xf, kohonen_weights, x_sq,
                                 w_sq.reshape(1, _H))
    winners = winners2d.reshape(_B)

    gather_k, n_chunks = _make_gather()
    table = jnp.pad(grossberg_weights.T, ((0, 0), (0, 128 - _O)))
    idx2d = winners.reshape(_B // 128, 128)
    output = gather_k(table, idx2d)[:, :_O]
    output = output.astype(jnp.bfloat16).astype(jnp.float32)
    return (output, winners, b)


# speedup vs baseline: 12.2309x; 1.2113x over previous
import functools

import jax
import jax.numpy as jnp
from jax import lax
from jax.experimental import pallas as pl
from jax.experimental.pallas import tpu as pltpu
from jax.experimental.pallas import tpu_sc as plsc

_B = 16384
_D = 32
_H = 8192
_O = 64

_BB = 512


_HC = _H // 2


def _round_bf16(v):
    u = lax.bitcast_convert_type(v, jnp.uint32)
    u = (u + 0x7FFF + ((u >> 16) & 1)) & jnp.uint32(0xFFFF0000)
    return lax.bitcast_convert_type(u, jnp.float32)


def _winners_body(x_ref, w_ref, xsq_ref, wsq_ref, win_ref):
    xb16 = x_ref[...].astype(jnp.bfloat16)
    x_sq = xsq_ref[...]
    iotaf = lax.broadcasted_iota(jnp.int32, (_HC, _BB), 0).astype(jnp.float32)
    mins, idxs = [], []
    for h in range(2):
        wc = w_ref[pl.ds(h * _HC, _HC), :]
        w_sq = wsq_ref[pl.ds(h * _HC, _HC), :]
        mm = lax.dot_general(wc.astype(jnp.bfloat16), xb16,
                             (((1,), (1,)), ((), ())),
                             preferred_element_type=jnp.float32)
        d2 = (x_sq + w_sq) - 2.0 * mm
        m2 = jnp.maximum(jnp.min(d2, axis=0, keepdims=True), 0.0)
        s = jnp.sqrt(m2)
        m2_bits = lax.bitcast_convert_type(m2, jnp.int32)
        hi = m2
        for k in range(1, 8):
            cand = lax.bitcast_convert_type(m2_bits + k, jnp.float32)
            hi = jnp.where(jnp.sqrt(cand) == s, cand, hi)
        idx = jnp.min(jnp.where(d2 <= hi, iotaf, float(_HC)), axis=0,
                      keepdims=True) + float(h * _HC)
        mins.append(s)
        idxs.append(idx)
    take1 = mins[1] < _round_bf16(mins[0])
    win = jnp.where(take1, idxs[1], idxs[0]).astype(jnp.int32)
    win_ref[...] = win.reshape(1, 1, _BB)


def _compute_winners(x, kohonen_weights, x_sq, w_sq):
    grid = (_B // _BB,)
    return pl.pallas_call(
        _winners_body,
        grid=grid,
        in_specs=[
            pl.BlockSpec((_BB, _D), lambda i: (i, 0)),
            pl.BlockSpec((_H, _D), lambda i: (0, 0)),
            pl.BlockSpec((1, _BB), lambda i: (0, i)),
            pl.BlockSpec((_H, 1), lambda i: (0, 0)),
        ],
        out_specs=pl.BlockSpec((1, 1, _BB), lambda i: (i, 0, 0)),
        out_shape=jax.ShapeDtypeStruct((_B // _BB, 1, _BB), jnp.int32),
    )(x, kohonen_weights, x_sq, w_sq)


def _make_gather():
    info = plsc.get_sparse_core_info()
    nc, ns = info.num_cores, info.num_subcores
    nw = nc * ns
    b_per_w = _B // nw
    n_chunks = b_per_w // 128
    mesh = plsc.VectorSubcoreMesh(core_axis_name="c", subcore_axis_name="s")

    @functools.partial(
        pl.kernel,
        mesh=mesh,
        out_type=jax.ShapeDtypeStruct((_B, 128), jnp.float32),
        scratch_types=[
            pltpu.VMEM((n_chunks, 128), jnp.int32),
            pltpu.VMEM((b_per_w, 128), jnp.float32),
            pltpu.SemaphoreType.DMA,
        ],
    )
    def gather_k(table_hbm, idx_hbm, out_hbm, idx_v, rows_v, sem):
        wid = lax.axis_index("s") * nc + lax.axis_index("c")
        base = wid * b_per_w
        pltpu.sync_copy(idx_hbm.at[pl.ds(wid * n_chunks, n_chunks)], idx_v)
        copies = [
            pltpu.async_copy(table_hbm.at[idx_v.at[j]],
                             rows_v.at[pl.ds(j * 128, 128)], sem)
            for j in range(n_chunks)
        ]
        for c in copies:
            c.wait()
        pltpu.sync_copy(rows_v, out_hbm.at[pl.ds(base, b_per_w)])

    return gather_k, n_chunks


def kernel(x, kohonen_weights, grossberg_weights):
    b = x.shape[0]
    xf = x.reshape(b, -1)
    x_sq = jnp.sum(xf * xf, axis=1, keepdims=True)
    w_sq = jnp.sum(kohonen_weights * kohonen_weights, axis=1)

    winners3d = _compute_winners(xf, kohonen_weights, x_sq.reshape(1, _B),
                                 w_sq.reshape(_H, 1))
    winners = winners3d.reshape(_B)

    gather_k, n_chunks = _make_gather()
    table = jnp.pad(grossberg_weights.T, ((0, 0), (0, 128 - _O)))
    idx2d = winners.reshape(_B // 128, 128)
    output = gather_k(table, idx2d)[:, :_O]
    output = output.astype(jnp.bfloat16).astype(jnp.float32)
    return (output, winners, b)
